# Initial kernel scaffold; baseline (speedup 1.0000x reference)
import jax
import jax.numpy as jnp
from jax.experimental import pallas as pl


def kernel(x, pe_table):
    raise NotImplementedError("write your pallas kernel here")


# speedup vs baseline: 3.1840x; 3.1840x over previous
import jax
import jax.numpy as jnp
from jax.experimental import pallas as pl

_ROWS = 512


def _body(x_ref, pe_ref, o_ref):
    dx = x_ref.shape[-1]
    o_ref[:, :, :dx] = x_ref[...]
    o_ref[:, :, dx:] = pe_ref[...][None]


def kernel(x, pe_table):
    b, size, dx = x.shape
    dim = pe_table.shape[-1]
    grid = (size // _ROWS, b)
    return pl.pallas_call(
        _body,
        grid=grid,
        in_specs=[
            pl.BlockSpec((1, _ROWS, dx), lambda i, bb: (bb, i, 0)),
            pl.BlockSpec((_ROWS, dim), lambda i, bb: (i, 0)),
        ],
        out_specs=pl.BlockSpec((1, _ROWS, dx + dim), lambda i, bb: (bb, i, 0)),
        out_shape=jax.ShapeDtypeStruct((b, size, dx + dim), x.dtype),
    )(x, pe_table)
